# Initial kernel scaffold; baseline (speedup 1.0000x reference)
import jax
import jax.numpy as jnp
from jax.experimental import pallas as pl


def kernel(x, edge_index, W1l, b1l, W1r, b1r, att1, bias1, W2l, b2l, W2r, b2r, att2, bias2):
    raise NotImplementedError("write your pallas kernel here")


# speedup vs baseline: 23.6245x; 23.6245x over previous
import functools

import jax
import jax.numpy as jnp
from jax import lax
from jax.experimental import pallas as pl
from jax.experimental.pallas import tpu as pltpu
from jax.experimental.pallas import tpu_sc as plsc

N = 10000
D = 256
HC = 64
NCLS = 7

NSC = 2
NS = 16
NW = NSC * NS
N_PAD = 10240

KC = 384
CH = 14
EW = KC * CH
ET_PAD = EW * NW
EROWS = ET_PAD // 128
RPT = EW // 128
SROWS = 56
EROWS_PAD = 1352

BE = 3584
NBE = ET_PAD // BE
BN = 1280
BO = 2000

_SC_PARAMS = pltpu.CompilerParams(use_tc_tiling_on_sc=False)


@functools.cache
def _sc_mesh():
    return plsc.VectorSubcoreMesh(
        core_axis_name="c", subcore_axis_name="s",
        num_cores=NSC, num_subcores=NS)


def _mm_body(x_ref, w_ref, b_ref, o_ref):
    o_ref[...] = (
        jnp.dot(x_ref[...], w_ref[...], preferred_element_type=jnp.float32)
        + b_ref[...])


def _project(x, wcat, bcat):
    return pl.pallas_call(
        _mm_body,
        out_shape=jax.ShapeDtypeStruct((N, 2 * HC), jnp.float32),
        grid=(5,),
        in_specs=[
            pl.BlockSpec((N // 5, D), lambda i: (i, 0)),
            pl.BlockSpec((D, 2 * HC), lambda i: (0, 0)),
            pl.BlockSpec((1, 2 * HC), lambda i: (0, 0)),
        ],
        out_specs=pl.BlockSpec((N // 5, 2 * HC), lambda i: (i, 0)),
    )(x, wcat, bcat)


def _logits_body(xls_ref, xrd_ref, atts_ref, lg_ref, mx_ref):
    e = xls_ref[...] + xrd_ref[...]
    e = jnp.where(e > 0, e, 0.2 * e)
    lg = jnp.dot(e, atts_ref[...], preferred_element_type=jnp.float32)
    lg_ref[...] = lg
    mx_ref[0, 0, :] = jnp.max(lg, axis=0)


def _edge_logits(xls, xrd, atts):
    dt = xls.shape[1]
    return pl.pallas_call(
        _logits_body,
        out_shape=(
            jax.ShapeDtypeStruct((ET_PAD, 8), jnp.float32),
            jax.ShapeDtypeStruct((NBE, 1, 8), jnp.float32),
        ),
        grid=(NBE,),
        in_specs=[
            pl.BlockSpec((BE, dt), lambda i: (i, 0)),
            pl.BlockSpec((BE, dt), lambda i: (i, 0)),
            pl.BlockSpec((dt, 8), lambda i: (0, 0)),
        ],
        out_specs=(
            pl.BlockSpec((BE, 8), lambda i: (i, 0)),
            pl.BlockSpec((1, 1, 8), lambda i: (i, 0, 0)),
        ),
    )(xls, xrd, atts)


def _msg1_body(lg_ref, xls_ref, mx_ref, st_ref, o_ref):
    a = jnp.exp(lg_ref[...] - mx_ref[0:1, :])
    arep = jnp.dot(a, st_ref[...], preferred_element_type=jnp.float32)
    m64 = arep * xls_ref[...]
    o_ref[...] = jnp.concatenate([m64, a, jnp.zeros_like(a)], axis=1)


def _edge_msg1(lg, xls, mx, st):
    return pl.pallas_call(
        _msg1_body,
        out_shape=jax.ShapeDtypeStruct((ET_PAD, 80), jnp.float32),
        grid=(NBE,),
        in_specs=[
            pl.BlockSpec((BE, 8), lambda i: (i, 0)),
            pl.BlockSpec((BE, HC), lambda i: (i, 0)),
            pl.BlockSpec((8, 8), lambda i: (0, 0)),
            pl.BlockSpec((8, HC), lambda i: (0, 0)),
        ],
        out_specs=pl.BlockSpec((BE, 80), lambda i: (i, 0)),
    )(lg, xls, mx, st)


def _msg2_body(lg_ref, hls_ref, mx_ref, o_ref):
    a = jnp.exp(lg_ref[...] - mx_ref[0:1, :])
    o_ref[...] = jnp.concatenate([a * hls_ref[:, 0:8], a], axis=1)


def _edge_msg2(lg, hls, mx):
    return pl.pallas_call(
        _msg2_body,
        out_shape=jax.ShapeDtypeStruct((ET_PAD, 16), jnp.float32),
        grid=(NBE,),
        in_specs=[
            pl.BlockSpec((BE, 8), lambda i: (i, 0)),
            pl.BlockSpec((BE, 16), lambda i: (i, 0)),
            pl.BlockSpec((8, 8), lambda i: (0, 0)),
        ],
        out_specs=pl.BlockSpec((BE, 16), lambda i: (i, 0)),
    )(lg, hls, mx)


def _comb1_body(a0_ref, a1_ref, st_ref, b1_ref, w2_ref, b2_ref,
                tl_ref, tr_ref):
    t = a0_ref[...] + a1_ref[...]
    s8 = t[:, 64:72]
    srep = jnp.dot(s8, st_ref[...], preferred_element_type=jnp.float32)
    h = t[:, 0:64] / srep + b1_ref[...]
    h = jnp.where(h > 0, h, jnp.exp(jnp.minimum(h, 0.0)) - 1.0)
    rows = (jax.lax.broadcasted_iota(jnp.int32, (BN, 1), 0)
            + pl.program_id(0) * BN)
    h = jnp.where(rows < N, h, 0.0)
    hp = jnp.dot(h, w2_ref[...], preferred_element_type=jnp.float32) + b2_ref[...]
    hp = jnp.where(rows < N, hp, 0.0)
    tl_ref[...] = hp[:, 0:16]
    tr_ref[...] = hp[:, 16:32]


def _combine1(a0, a1, st, b1, w2cat, b2cat):
    return pl.pallas_call(
        _comb1_body,
        out_shape=(
            jax.ShapeDtypeStruct((N_PAD, 16), jnp.float32),
            jax.ShapeDtypeStruct((N_PAD, 16), jnp.float32),
        ),
        grid=(N_PAD // BN,),
        in_specs=[
            pl.BlockSpec((BN, 80), lambda i: (i, 0)),
            pl.BlockSpec((BN, 80), lambda i: (i, 0)),
            pl.BlockSpec((8, HC), lambda i: (0, 0)),
            pl.BlockSpec((1, HC), lambda i: (0, 0)),
            pl.BlockSpec((HC, 32), lambda i: (0, 0)),
            pl.BlockSpec((1, 32), lambda i: (0, 0)),
        ],
        out_specs=(
            pl.BlockSpec((BN, 16), lambda i: (i, 0)),
            pl.BlockSpec((BN, 16), lambda i: (i, 0)),
        ),
    )(a0, a1, st, b1, w2cat, b2cat)


def _out_body(a0_ref, a1_ref, b2_ref, o_ref):
    t = a0_ref[...] + a1_ref[...]
    s = t[:, 8:9]
    o = t[:, 0:7] / s + b2_ref[...]
    m = jnp.max(o, axis=1, keepdims=True)
    p = jnp.exp(o - m)
    o_ref[...] = p / jnp.sum(p, axis=1, keepdims=True)


def _final_out(a0, a1, b2):
    return pl.pallas_call(
        _out_body,
        out_shape=jax.ShapeDtypeStruct((N, NCLS), jnp.float32),
        grid=(N // BO,),
        in_specs=[
            pl.BlockSpec((BO, 16), lambda i: (i, 0)),
            pl.BlockSpec((BO, 16), lambda i: (i, 0)),
            pl.BlockSpec((1, NCLS), lambda i: (0, 0)),
        ],
        out_specs=pl.BlockSpec((BO, NCLS), lambda i: (i, 0)),
    )(a0, a1, b2)


def _gather_pairs(src2d, dst2d, tl, tr):
    dt = tl.shape[1]

    @functools.partial(
        pl.kernel,
        out_type=(
            jax.ShapeDtypeStruct((ET_PAD, dt), jnp.float32),
            jax.ShapeDtypeStruct((ET_PAD, dt), jnp.float32),
        ),
        mesh=_sc_mesh(),
        compiler_params=_SC_PARAMS,
        scratch_types=[
            pltpu.VMEM((SROWS, 128), jnp.int32),
            pltpu.VMEM((SROWS, 128), jnp.int32),
            pltpu.VMEM((KC, dt), jnp.float32),
            pltpu.VMEM((KC, dt), jnp.float32),
            pltpu.SemaphoreType.DMA,
            pltpu.SemaphoreType.DMA,
        ],
    )
    def body(src_hbm, dst_hbm, tl_hbm, tr_hbm, outl_hbm, outr_hbm,
             sidx, didx, bufl, bufr, sem1, sem2):
        c = lax.axis_index("c")
        s = lax.axis_index("s")
        wid = s * NSC + c
        base8 = pl.multiple_of((wid * RPT) // 8 * 8, 8)
        o = wid * RPT - base8
        pltpu.sync_copy(src_hbm.at[pl.ds(base8, SROWS)], sidx)
        pltpu.sync_copy(dst_hbm.at[pl.ds(base8, SROWS)], didx)
        for j in range(CH):
            cps = []
            for i in range(3):
                r = o + j * 3 + i
                cps.append(pltpu.async_copy(
                    tl_hbm.at[sidx.at[r]], bufl.at[pl.ds(i * 128, 128)], sem1))
                cps.append(pltpu.async_copy(
                    tr_hbm.at[didx.at[r]], bufr.at[pl.ds(i * 128, 128)], sem2))
            for cp in cps:
                cp.wait()
            off = pl.multiple_of(wid * EW + j * KC, 8)
            pltpu.sync_copy(bufl, outl_hbm.at[pl.ds(off, KC)])
            pltpu.sync_copy(bufr, outr_hbm.at[pl.ds(off, KC)])

    return body(src2d, dst2d, tl, tr)


def _scatter_add(dst2d, msg):
    w = msg.shape[1]
    rt = N_PAD // NS
    zb = 64

    @functools.partial(
        pl.kernel,
        out_type=jax.ShapeDtypeStruct((NSC, N_PAD, w), jnp.float32),
        mesh=_sc_mesh(),
        compiler_params=_SC_PARAMS,
        scratch_types=[
            pltpu.VMEM_SHARED((N_PAD, w), jnp.float32),
            pltpu.VMEM((zb, w), jnp.float32),
            pltpu.VMEM((KC, w), jnp.float32),
            pltpu.VMEM((SROWS, 128), jnp.int32),
        ],
    )
    def body(dst_hbm, msg_hbm, out_hbm, acc, zbuf, mbuf, didx):
        c = lax.axis_index("c")
        s = lax.axis_index("s")
        wid = s * NSC + c
        base8 = pl.multiple_of((wid * RPT) // 8 * 8, 8)
        o = wid * RPT - base8
        pltpu.sync_copy(dst_hbm.at[pl.ds(base8, SROWS)], didx)

        def zero_row(i, carry):
            for k in range(w // 16):
                zbuf[i, pl.ds(k * 16, 16)] = jnp.zeros((16,), jnp.float32)
            return carry
        lax.fori_loop(0, zb, zero_row, 0)
        row0 = pl.multiple_of(s * rt, 8)
        for r in range(rt // zb):
            pltpu.sync_copy(zbuf, acc.at[pl.ds(row0 + r * zb, zb)])
        plsc.subcore_barrier()

        for j in range(CH):
            off = pl.multiple_of(wid * EW + j * KC, 8)
            pltpu.sync_copy(msg_hbm.at[pl.ds(off, KC)], mbuf)
            for i in range(3):
                pltpu.sync_copy(mbuf.at[pl.ds(i * 128, 128)],
                                acc.at[didx.at[o + j * 3 + i]], add=True)
        plsc.subcore_barrier()
        pltpu.sync_copy(acc.at[pl.ds(row0, rt)],
                        out_hbm.at[c, pl.ds(row0, rt)])

    return body(dst2d, msg)


def kernel(x, edge_index, W1l, b1l, W1r, b1r, att1, bias1,
           W2l, b2l, W2r, b2r, att2, bias2):
    ei = edge_index.astype(jnp.int32)
    loop = jnp.arange(N, dtype=jnp.int32)
    npad = EROWS_PAD * 128 - (ei.shape[1] + N)
    src = jnp.concatenate([ei[0], loop, jnp.zeros((npad,), jnp.int32)])
    dst = jnp.concatenate([ei[1], loop, jnp.full((npad,), N, jnp.int32)])
    src2d = src.reshape(EROWS_PAD, 128)
    dst2d = dst.reshape(EROWS_PAD, 128)

    wcat = jnp.concatenate([W1l, W1r], axis=1)
    bcat = jnp.concatenate([b1l, b1r]).reshape(1, 2 * HC)
    proj = _project(x, wcat, bcat)
    xl = proj[:, :HC]
    xr_pad = jnp.concatenate(
        [proj[:, HC:], jnp.zeros((N_PAD - N, HC), jnp.float32)], axis=0)

    sel = (jnp.arange(HC)[:, None] // 8 == jnp.arange(8)[None, :])
    sel = sel.astype(jnp.float32)
    st1 = sel.T

    xls, xrd = _gather_pairs(src2d, dst2d, xl, xr_pad)
    atts1 = att1.reshape(HC)[:, None] * sel
    lg1, bmax1 = _edge_logits(xls, xrd, atts1)
    mx1 = jnp.broadcast_to(jnp.max(bmax1, axis=(0, 1))[None, :], (8, 8))
    msg1 = _edge_msg1(lg1, xls, mx1, st1)
    acc1 = _scatter_add(dst2d, msg1)

    w2cat = jnp.zeros((HC, 32), jnp.float32)
    w2cat = w2cat.at[:, 0:NCLS].set(W2l).at[:, 16:16 + NCLS].set(W2r)
    b2cat = jnp.zeros((32,), jnp.float32)
    b2cat = b2cat.at[0:NCLS].set(b2l).at[16:16 + NCLS].set(b2r)
    tl2, tr2 = _combine1(acc1[0], acc1[1], st1, bias1.reshape(1, HC),
                         w2cat, b2cat.reshape(1, 32))

    hls, hrd = _gather_pairs(src2d, dst2d, tl2, tr2)
    att2p = jnp.zeros((16,), jnp.float32).at[0:NCLS].set(att2.reshape(NCLS))
    atts2 = jnp.broadcast_to(att2p[:, None], (16, 8))
    lg2, bmax2 = _edge_logits(hls, hrd, atts2)
    mx2 = jnp.broadcast_to(jnp.max(bmax2, axis=(0, 1))[None, :], (8, 8))
    msg2 = _edge_msg2(lg2, hls, mx2)
    acc2 = _scatter_add(dst2d, msg2)

    return _final_out(acc2[0], acc2[1], bias2.reshape(1, NCLS))


# speedup vs baseline: 29.7429x; 1.2590x over previous
import functools

import jax
import jax.numpy as jnp
from jax import lax
from jax.experimental import pallas as pl
from jax.experimental.pallas import tpu as pltpu
from jax.experimental.pallas import tpu_sc as plsc

N = 10000
D = 256
HC = 64
NCLS = 7

NSC = 2
NS = 16
NW = NSC * NS
N_PAD = 10240

KC = 384
CH = 14
EW = KC * CH
ET_PAD = EW * NW
EROWS = ET_PAD // 128
RPT = EW // 128
SROWS = 56
EROWS_PAD = 1352

BE = 3584
NBE = ET_PAD // BE
BN = 1280
BO = 2000

_SC_PARAMS = pltpu.CompilerParams(use_tc_tiling_on_sc=False)


@functools.cache
def _sc_mesh():
    return plsc.VectorSubcoreMesh(
        core_axis_name="c", subcore_axis_name="s",
        num_cores=NSC, num_subcores=NS)


def _mm_body(x_ref, w_ref, b_ref, o_ref, mn_ref, mx_ref):
    o = (jnp.dot(x_ref[...], w_ref[...], preferred_element_type=jnp.float32)
         + b_ref[...])
    o_ref[...] = o
    cmn = jnp.broadcast_to(jnp.min(o, axis=0, keepdims=True), (8, 2 * HC))
    cmx = jnp.broadcast_to(jnp.max(o, axis=0, keepdims=True), (8, 2 * HC))
    i = pl.program_id(0)

    @pl.when(i == 0)
    def _():
        mn_ref[...] = cmn
        mx_ref[...] = cmx

    @pl.when(i > 0)
    def _():
        mn_ref[...] = jnp.minimum(mn_ref[...], cmn)
        mx_ref[...] = jnp.maximum(mx_ref[...], cmx)


def _project(x, wcat, bcat):
    return pl.pallas_call(
        _mm_body,
        out_shape=(
            jax.ShapeDtypeStruct((N, 2 * HC), jnp.float32),
            jax.ShapeDtypeStruct((8, 2 * HC), jnp.float32),
            jax.ShapeDtypeStruct((8, 2 * HC), jnp.float32),
        ),
        grid=(5,),
        in_specs=[
            pl.BlockSpec((N // 5, D), lambda i: (i, 0)),
            pl.BlockSpec((D, 2 * HC), lambda i: (0, 0)),
            pl.BlockSpec((1, 2 * HC), lambda i: (0, 0)),
        ],
        out_specs=(
            pl.BlockSpec((N // 5, 2 * HC), lambda i: (i, 0)),
            pl.BlockSpec((8, 2 * HC), lambda i: (0, 0)),
            pl.BlockSpec((8, 2 * HC), lambda i: (0, 0)),
        ),
    )(x, wcat, bcat)


def _fused1_body(xls_ref, xrd_ref, atts_ref, mx_ref, st_ref, o_ref):
    e = xls_ref[...] + xrd_ref[...]
    e = jnp.where(e > 0, e, 0.2 * e)
    lg = jnp.dot(e, atts_ref[...], preferred_element_type=jnp.float32)
    a = jnp.exp(lg - mx_ref[0:1, :])
    arep = jnp.dot(a, st_ref[...], preferred_element_type=jnp.float32)
    m64 = arep * xls_ref[...]
    o_ref[...] = jnp.concatenate([m64, a, jnp.zeros_like(a)], axis=1)


def _edge_fused1(xls, xrd, atts, mx, st):
    return pl.pallas_call(
        _fused1_body,
        out_shape=jax.ShapeDtypeStruct((ET_PAD, 80), jnp.float32),
        grid=(NBE,),
        in_specs=[
            pl.BlockSpec((BE, HC), lambda i: (i, 0)),
            pl.BlockSpec((BE, HC), lambda i: (i, 0)),
            pl.BlockSpec((HC, 8), lambda i: (0, 0)),
            pl.BlockSpec((8, 8), lambda i: (0, 0)),
            pl.BlockSpec((8, HC), lambda i: (0, 0)),
        ],
        out_specs=pl.BlockSpec((BE, 80), lambda i: (i, 0)),
    )(xls, xrd, atts, mx, st)


def _fused2_body(hls_ref, hrd_ref, atts_ref, mx_ref, o_ref):
    e = hls_ref[...] + hrd_ref[...]
    e = jnp.where(e > 0, e, 0.2 * e)
    lg = jnp.dot(e, atts_ref[...], preferred_element_type=jnp.float32)
    a = jnp.exp(lg - mx_ref[0:1, :])
    o_ref[...] = jnp.concatenate([a * hls_ref[:, 0:8], a], axis=1)


def _edge_fused2(hls, hrd, atts, mx):
    return pl.pallas_call(
        _fused2_body,
        out_shape=jax.ShapeDtypeStruct((ET_PAD, 16), jnp.float32),
        grid=(NBE,),
        in_specs=[
            pl.BlockSpec((BE, 16), lambda i: (i, 0)),
            pl.BlockSpec((BE, 16), lambda i: (i, 0)),
            pl.BlockSpec((16, 8), lambda i: (0, 0)),
            pl.BlockSpec((8, 8), lambda i: (0, 0)),
        ],
        out_specs=pl.BlockSpec((BE, 16), lambda i: (i, 0)),
    )(hls, hrd, atts, mx)


def _comb1_body(a0_ref, a1_ref, st_ref, b1_ref, w2_ref, b2_ref,
                tl_ref, tr_ref, mn_ref, mx_ref):
    t = a0_ref[...] + a1_ref[...]
    s8 = t[:, 64:72]
    srep = jnp.dot(s8, st_ref[...], preferred_element_type=jnp.float32)
    h = t[:, 0:64] / srep + b1_ref[...]
    h = jnp.where(h > 0, h, jnp.exp(jnp.minimum(h, 0.0)) - 1.0)
    rows = (jax.lax.broadcasted_iota(jnp.int32, (BN, 1), 0)
            + pl.program_id(0) * BN)
    h = jnp.where(rows < N, h, 0.0)
    hp = jnp.dot(h, w2_ref[...], preferred_element_type=jnp.float32) + b2_ref[...]
    hp = jnp.where(rows < N, hp, 0.0)
    tl_ref[...] = hp[:, 0:16]
    tr_ref[...] = hp[:, 16:32]
    cmn = jnp.broadcast_to(jnp.min(hp, axis=0, keepdims=True), (8, 32))
    cmx = jnp.broadcast_to(jnp.max(hp, axis=0, keepdims=True), (8, 32))
    i = pl.program_id(0)

    @pl.when(i == 0)
    def _():
        mn_ref[...] = cmn
        mx_ref[...] = cmx

    @pl.when(i > 0)
    def _():
        mn_ref[...] = jnp.minimum(mn_ref[...], cmn)
        mx_ref[...] = jnp.maximum(mx_ref[...], cmx)


def _combine1(a0, a1, st, b1, w2cat, b2cat):
    return pl.pallas_call(
        _comb1_body,
        out_shape=(
            jax.ShapeDtypeStruct((N_PAD, 16), jnp.float32),
            jax.ShapeDtypeStruct((N_PAD, 16), jnp.float32),
            jax.ShapeDtypeStruct((8, 32), jnp.float32),
            jax.ShapeDtypeStruct((8, 32), jnp.float32),
        ),
        grid=(N_PAD // BN,),
        in_specs=[
            pl.BlockSpec((BN, 80), lambda i: (i, 0)),
            pl.BlockSpec((BN, 80), lambda i: (i, 0)),
            pl.BlockSpec((8, HC), lambda i: (0, 0)),
            pl.BlockSpec((1, HC), lambda i: (0, 0)),
            pl.BlockSpec((HC, 32), lambda i: (0, 0)),
            pl.BlockSpec((1, 32), lambda i: (0, 0)),
        ],
        out_specs=(
            pl.BlockSpec((BN, 16), lambda i: (i, 0)),
            pl.BlockSpec((BN, 16), lambda i: (i, 0)),
            pl.BlockSpec((8, 32), lambda i: (0, 0)),
            pl.BlockSpec((8, 32), lambda i: (0, 0)),
        ),
    )(a0, a1, st, b1, w2cat, b2cat)


def _out_body(a0_ref, a1_ref, b2_ref, o_ref):
    t = a0_ref[...] + a1_ref[...]
    s = t[:, 8:9]
    o = t[:, 0:7] / s + b2_ref[...]
    m = jnp.max(o, axis=1, keepdims=True)
    p = jnp.exp(o - m)
    o_ref[...] = p / jnp.sum(p, axis=1, keepdims=True)


def _final_out(a0, a1, b2):
    return pl.pallas_call(
        _out_body,
        out_shape=jax.ShapeDtypeStruct((N, NCLS), jnp.float32),
        grid=(N // BO,),
        in_specs=[
            pl.BlockSpec((BO, 16), lambda i: (i, 0)),
            pl.BlockSpec((BO, 16), lambda i: (i, 0)),
            pl.BlockSpec((1, NCLS), lambda i: (0, 0)),
        ],
        out_specs=pl.BlockSpec((BO, NCLS), lambda i: (i, 0)),
    )(a0, a1, b2)


def _gather_pairs(src2d, dst2d, tl, tr):
    dt = tl.shape[1]

    @functools.partial(
        pl.kernel,
        out_type=(
            jax.ShapeDtypeStruct((ET_PAD, dt), jnp.float32),
            jax.ShapeDtypeStruct((ET_PAD, dt), jnp.float32),
        ),
        mesh=_sc_mesh(),
        compiler_params=_SC_PARAMS,
        scratch_types=[
            pltpu.VMEM((SROWS, 128), jnp.int32),
            pltpu.VMEM((SROWS, 128), jnp.int32),
            pltpu.VMEM((KC, dt), jnp.float32),
            pltpu.VMEM((KC, dt), jnp.float32),
            pltpu.VMEM((KC, dt), jnp.float32),
            pltpu.VMEM((KC, dt), jnp.float32),
            pltpu.SemaphoreType.DMA,
            pltpu.SemaphoreType.DMA,
            pltpu.SemaphoreType.DMA,
            pltpu.SemaphoreType.DMA,
        ],
    )
    def body(src_hbm, dst_hbm, tl_hbm, tr_hbm, outl_hbm, outr_hbm,
             sidx, didx, bufl0, bufl1, bufr0, bufr1,
             seml0, seml1, semr0, semr1):
        c = lax.axis_index("c")
        s = lax.axis_index("s")
        wid = s * NSC + c
        base8 = pl.multiple_of((wid * RPT) // 8 * 8, 8)
        o = wid * RPT - base8
        pltpu.sync_copy(src_hbm.at[pl.ds(base8, SROWS)], sidx)
        pltpu.sync_copy(dst_hbm.at[pl.ds(base8, SROWS)], didx)
        bufls, bufrs = [bufl0, bufl1], [bufr0, bufr1]
        semls, semrs = [seml0, seml1], [semr0, semr1]

        def fire(j, p):
            cps = []
            for i in range(3):
                r = o + j * 3 + i
                cps.append(pltpu.async_copy(
                    tl_hbm.at[sidx.at[r]],
                    bufls[p].at[pl.ds(i * 128, 128)], semls[p]))
                cps.append(pltpu.async_copy(
                    tr_hbm.at[didx.at[r]],
                    bufrs[p].at[pl.ds(i * 128, 128)], semrs[p]))
            return cps

        infl = {0: fire(0, 0), 1: fire(1, 1)}
        for j in range(CH):
            p = j % 2
            for cp in infl[p]:
                cp.wait()
            off = pl.multiple_of(wid * EW + j * KC, 8)
            pltpu.sync_copy(bufls[p], outl_hbm.at[pl.ds(off, KC)])
            pltpu.sync_copy(bufrs[p], outr_hbm.at[pl.ds(off, KC)])
            if j + 2 < CH:
                infl[p] = fire(j + 2, p)

    return body(src2d, dst2d, tl, tr)


def _scatter_add(dst2d, msg):
    w = msg.shape[1]
    rt = N_PAD // NS
    zb = 64

    @functools.partial(
        pl.kernel,
        out_type=jax.ShapeDtypeStruct((NSC, N_PAD, w), jnp.float32),
        mesh=_sc_mesh(),
        compiler_params=_SC_PARAMS,
        scratch_types=[
            pltpu.VMEM_SHARED((N_PAD, w), jnp.float32),
            pltpu.VMEM((zb, w), jnp.float32),
            pltpu.VMEM((KC, w), jnp.float32),
            pltpu.VMEM((KC, w), jnp.float32),
            pltpu.VMEM((SROWS, 128), jnp.int32),
            pltpu.SemaphoreType.DMA,
            pltpu.SemaphoreType.DMA,
        ],
    )
    def body(dst_hbm, msg_hbm, out_hbm, acc, zbuf, mbuf0, mbuf1, didx,
             semm0, semm1):
        c = lax.axis_index("c")
        s = lax.axis_index("s")
        wid = s * NSC + c
        base8 = pl.multiple_of((wid * RPT) // 8 * 8, 8)
        o = wid * RPT - base8
        pltpu.sync_copy(dst_hbm.at[pl.ds(base8, SROWS)], didx)
        mbufs, semms = [mbuf0, mbuf1], [semm0, semm1]

        def fire(j, p):
            off = pl.multiple_of(wid * EW + j * KC, 8)
            return pltpu.async_copy(
                msg_hbm.at[pl.ds(off, KC)], mbufs[p], semms[p])

        infl = {0: fire(0, 0), 1: fire(1, 1)}

        def zero_row(i, carry):
            for k in range(w // 16):
                zbuf[i, pl.ds(k * 16, 16)] = jnp.zeros((16,), jnp.float32)
            return carry
        lax.fori_loop(0, zb, zero_row, 0)
        row0 = pl.multiple_of(s * rt, 8)
        for r in range(rt // zb):
            pltpu.sync_copy(zbuf, acc.at[pl.ds(row0 + r * zb, zb)])
        plsc.subcore_barrier()

        for j in range(CH):
            p = j % 2
            infl[p].wait()
            for i in range(3):
                pltpu.sync_copy(mbufs[p].at[pl.ds(i * 128, 128)],
                                acc.at[didx.at[o + j * 3 + i]], add=True)
            if j + 2 < CH:
                infl[p] = fire(j + 2, p)
        plsc.subcore_barrier()
        pltpu.sync_copy(acc.at[pl.ds(row0, rt)],
                        out_hbm.at[c, pl.ds(row0, rt)])

    return body(dst2d, msg)


def kernel(x, edge_index, W1l, b1l, W1r, b1r, att1, bias1,
           W2l, b2l, W2r, b2r, att2, bias2):
    ei = edge_index.astype(jnp.int32)
    loop = jnp.arange(N, dtype=jnp.int32)
    npad = EROWS_PAD * 128 - (ei.shape[1] + N)
    src = jnp.concatenate([ei[0], loop, jnp.zeros((npad,), jnp.int32)])
    dst = jnp.concatenate([ei[1], loop, jnp.full((npad,), N, jnp.int32)])
    src2d = src.reshape(EROWS_PAD, 128)
    dst2d = dst.reshape(EROWS_PAD, 128)

    wcat = jnp.concatenate([W1l, W1r], axis=1)
    bcat = jnp.concatenate([b1l, b1r]).reshape(1, 2 * HC)
    proj, pmn, pmx = _project(x, wcat, bcat)
    xl = proj[:, :HC]
    xr_pad = jnp.concatenate(
        [proj[:, HC:], jnp.zeros((N_PAD - N, HC), jnp.float32)], axis=0)

    sel = (jnp.arange(HC)[:, None] // 8 == jnp.arange(8)[None, :])
    sel = sel.astype(jnp.float32)
    st1 = sel.T

    def lrelu(z):
        return jnp.where(z > 0, z, 0.2 * z)

    af1 = att1.reshape(HC)
    lo1 = lrelu(pmn[0, :HC] + pmn[0, HC:])
    hi1 = lrelu(pmx[0, :HC] + pmx[0, HC:])
    m1 = jnp.maximum(af1 * hi1, af1 * lo1) @ sel
    mx1 = jnp.broadcast_to(m1[None, :], (8, 8))

    xls, xrd = _gather_pairs(src2d, dst2d, xl, xr_pad)
    atts1 = af1[:, None] * sel
    msg1 = _edge_fused1(xls, xrd, atts1, mx1, st1)
    acc1 = _scatter_add(dst2d, msg1)

    w2cat = jnp.zeros((HC, 32), jnp.float32)
    w2cat = w2cat.at[:, 0:NCLS].set(W2l).at[:, 16:16 + NCLS].set(W2r)
    b2cat = jnp.zeros((32,), jnp.float32)
    b2cat = b2cat.at[0:NCLS].set(b2l).at[16:16 + NCLS].set(b2r)
    tl2, tr2, cmn, cmx = _combine1(acc1[0], acc1[1], st1,
                                   bias1.reshape(1, HC), w2cat,
                                   b2cat.reshape(1, 32))

    att2p = jnp.zeros((16,), jnp.float32).at[0:NCLS].set(att2.reshape(NCLS))
    lo2 = lrelu(cmn[0, 0:16] + cmn[0, 16:32])
    hi2 = lrelu(cmx[0, 0:16] + cmx[0, 16:32])
    m2 = jnp.sum(jnp.maximum(att2p * hi2, att2p * lo2))
    mx2 = jnp.full((8, 8), m2, jnp.float32)

    hls, hrd = _gather_pairs(src2d, dst2d, tl2, tr2)
    atts2 = jnp.broadcast_to(att2p[:, None], (16, 8))
    msg2 = _edge_fused2(hls, hrd, atts2, mx2)
    acc2 = _scatter_add(dst2d, msg2)

    return _final_out(acc2[0], acc2[1], bias2.reshape(1, NCLS))


# speedup vs baseline: 59.7110x; 2.0076x over previous
import functools

import jax
import jax.numpy as jnp
from jax import lax
from jax.experimental import pallas as pl
from jax.experimental.pallas import tpu as pltpu
from jax.experimental.pallas import tpu_sc as plsc

N = 10000
D = 256
HC = 64
NCLS = 7

NSC = 2
NS = 16
NW = NSC * NS
N_PAD = 10112

KC = 128
CH = 42
KR = KC // 128
EW = KC * CH
ET_PAD = EW * NW
EROWS = ET_PAD // 128
RPT = EW // 128
SROWS = 56
EROWS_PAD = 1352

BE = 3584
NBE = ET_PAD // BE
BN = 1264
BO = 2000

_SC_PARAMS = pltpu.CompilerParams(use_tc_tiling_on_sc=False)


def _take16(v, idx):
    return lax.gather(
        v, idx[:, None],
        dimension_numbers=lax.GatherDimensionNumbers(
            offset_dims=(), collapsed_slice_dims=(0,), start_index_map=(0,)),
        slice_sizes=(1,),
        mode=lax.GatherScatterMode.PROMISE_IN_BOUNDS)


@functools.cache
def _sc_mesh():
    return plsc.VectorSubcoreMesh(
        core_axis_name="c", subcore_axis_name="s",
        num_cores=NSC, num_subcores=NS)


def _mm_body(x_ref, w_ref, b_ref, o_ref, mn_ref, mx_ref):
    o = (jnp.dot(x_ref[...], w_ref[...], preferred_element_type=jnp.float32)
         + b_ref[...])
    o_ref[...] = o
    cmn = jnp.broadcast_to(jnp.min(o, axis=0, keepdims=True), (8, 2 * HC))
    cmx = jnp.broadcast_to(jnp.max(o, axis=0, keepdims=True), (8, 2 * HC))
    i = pl.program_id(0)

    @pl.when(i == 0)
    def _():
        mn_ref[...] = cmn
        mx_ref[...] = cmx

    @pl.when(i > 0)
    def _():
        mn_ref[...] = jnp.minimum(mn_ref[...], cmn)
        mx_ref[...] = jnp.maximum(mx_ref[...], cmx)


def _project(x, wcat, bcat):
    return pl.pallas_call(
        _mm_body,
        out_shape=(
            jax.ShapeDtypeStruct((N, 2 * HC), jnp.float32),
            jax.ShapeDtypeStruct((8, 2 * HC), jnp.float32),
            jax.ShapeDtypeStruct((8, 2 * HC), jnp.float32),
        ),
        grid=(5,),
        in_specs=[
            pl.BlockSpec((N // 5, D), lambda i: (i, 0)),
            pl.BlockSpec((D, 2 * HC), lambda i: (0, 0)),
            pl.BlockSpec((1, 2 * HC), lambda i: (0, 0)),
        ],
        out_specs=(
            pl.BlockSpec((N // 5, 2 * HC), lambda i: (i, 0)),
            pl.BlockSpec((8, 2 * HC), lambda i: (0, 0)),
            pl.BlockSpec((8, 2 * HC), lambda i: (0, 0)),
        ),
    )(x, wcat, bcat)


def _fused1_body(xlr_ref, atts_ref, mx_ref, st_ref, o_ref):
    xls = xlr_ref[:, 0:HC]
    e = xls + xlr_ref[:, HC:2 * HC]
    e = jnp.where(e > 0, e, 0.2 * e)
    lg = jnp.dot(e, atts_ref[...], preferred_element_type=jnp.float32)
    a = jnp.exp(lg - mx_ref[0:1, :])
    arep = jnp.dot(a, st_ref[...], preferred_element_type=jnp.float32)
    o_ref[...] = jnp.concatenate([arep * xls, arep], axis=1)


def _edge_fused1(xlr, atts, mx, st):
    return pl.pallas_call(
        _fused1_body,
        out_shape=jax.ShapeDtypeStruct((ET_PAD, 2 * HC), jnp.float32),
        grid=(NBE,),
        in_specs=[
            pl.BlockSpec((BE, 2 * HC), lambda i: (i, 0)),
            pl.BlockSpec((HC, 8), lambda i: (0, 0)),
            pl.BlockSpec((8, 8), lambda i: (0, 0)),
            pl.BlockSpec((8, HC), lambda i: (0, 0)),
        ],
        out_specs=pl.BlockSpec((BE, 2 * HC), lambda i: (i, 0)),
    )(xlr, atts, mx, st)


def _comb1_body(a0_ref, a1_ref, b1_ref, w2_ref, b2_ref,
                tl_ref, tr_ref, mn_ref, mx_ref):
    t = a0_ref[...] + a1_ref[...]
    h = t[:, 0:HC] / t[:, HC:2 * HC] + b1_ref[...]
    h = jnp.where(h > 0, h, jnp.exp(jnp.minimum(h, 0.0)) - 1.0)
    rows = (jax.lax.broadcasted_iota(jnp.int32, (BN, 1), 0)
            + pl.program_id(0) * BN)
    h = jnp.where(rows < N, h, 0.0)
    hp = jnp.dot(h, w2_ref[...], preferred_element_type=jnp.float32) + b2_ref[...]
    hp = jnp.where(rows < N, hp, 0.0)
    tl_ref[...] = hp[:, 0:16]
    tr_ref[...] = hp[:, 16:32]
    cmn = jnp.broadcast_to(jnp.min(hp, axis=0, keepdims=True), (8, 32))
    cmx = jnp.broadcast_to(jnp.max(hp, axis=0, keepdims=True), (8, 32))
    i = pl.program_id(0)

    @pl.when(i == 0)
    def _():
        mn_ref[...] = cmn
        mx_ref[...] = cmx

    @pl.when(i > 0)
    def _():
        mn_ref[...] = jnp.minimum(mn_ref[...], cmn)
        mx_ref[...] = jnp.maximum(mx_ref[...], cmx)


def _combine1(a0, a1, b1, w2cat, b2cat):
    return pl.pallas_call(
        _comb1_body,
        out_shape=(
            jax.ShapeDtypeStruct((N_PAD, 16), jnp.float32),
            jax.ShapeDtypeStruct((N_PAD, 16), jnp.float32),
            jax.ShapeDtypeStruct((8, 32), jnp.float32),
            jax.ShapeDtypeStruct((8, 32), jnp.float32),
        ),
        grid=(N_PAD // BN,),
        in_specs=[
            pl.BlockSpec((BN, 2 * HC), lambda i: (i, 0)),
            pl.BlockSpec((BN, 2 * HC), lambda i: (i, 0)),
            pl.BlockSpec((1, HC), lambda i: (0, 0)),
            pl.BlockSpec((HC, 32), lambda i: (0, 0)),
            pl.BlockSpec((1, 32), lambda i: (0, 0)),
        ],
        out_specs=(
            pl.BlockSpec((BN, 16), lambda i: (i, 0)),
            pl.BlockSpec((BN, 16), lambda i: (i, 0)),
            pl.BlockSpec((8, 32), lambda i: (0, 0)),
            pl.BlockSpec((8, 32), lambda i: (0, 0)),
        ),
    )(a0, a1, b1, w2cat, b2cat)


def _out_body(a0_ref, a1_ref, b2_ref, o_ref):
    t = a0_ref[...] + a1_ref[...]
    s = t[:, 8:9]
    o = t[:, 0:7] / s + b2_ref[...]
    m = jnp.max(o, axis=1, keepdims=True)
    p = jnp.exp(o - m)
    o_ref[...] = p / jnp.sum(p, axis=1, keepdims=True)


def _final_out(a0, a1, b2):
    return pl.pallas_call(
        _out_body,
        out_shape=jax.ShapeDtypeStruct((N, NCLS), jnp.float32),
        grid=(N // BO,),
        in_specs=[
            pl.BlockSpec((BO, 16), lambda i: (i, 0)),
            pl.BlockSpec((BO, 16), lambda i: (i, 0)),
            pl.BlockSpec((1, NCLS), lambda i: (0, 0)),
        ],
        out_specs=pl.BlockSpec((BO, NCLS), lambda i: (i, 0)),
    )(a0, a1, b2)


def _gather_pairs_combined(src2d, dst2d, tl, tr):
    dt = tl.shape[1]

    @functools.partial(
        pl.kernel,
        out_type=jax.ShapeDtypeStruct((ET_PAD, 2 * dt), jnp.float32),
        mesh=_sc_mesh(),
        compiler_params=_SC_PARAMS,
        scratch_types=[
            pltpu.VMEM((SROWS, 128), jnp.int32),
            pltpu.VMEM((SROWS, 128), jnp.int32),
            pltpu.VMEM((KC, dt), jnp.float32),
            pltpu.VMEM((KC, dt), jnp.float32),
            pltpu.VMEM((KC, dt), jnp.float32),
            pltpu.VMEM((KC, dt), jnp.float32),
            pltpu.SemaphoreType.DMA,
            pltpu.SemaphoreType.DMA,
            pltpu.SemaphoreType.DMA,
            pltpu.SemaphoreType.DMA,
        ],
    )
    def body(src_hbm, dst_hbm, tl_hbm, tr_hbm, out_hbm,
             sidx, didx, bufl0, bufl1, bufr0, bufr1,
             seml0, seml1, semr0, semr1):
        c = lax.axis_index("c")
        s = lax.axis_index("s")
        wid = s * NSC + c
        base8 = pl.multiple_of((wid * RPT) // 8 * 8, 8)
        o = wid * RPT - base8
        pltpu.sync_copy(src_hbm.at[pl.ds(base8, SROWS)], sidx)
        pltpu.sync_copy(dst_hbm.at[pl.ds(base8, SROWS)], didx)
        bufls, bufrs = [bufl0, bufl1], [bufr0, bufr1]
        semls, semrs = [seml0, seml1], [semr0, semr1]

        def fire(j, p):
            cps = []
            for i in range(KR):
                r = o + j * KR + i
                cps.append(pltpu.async_copy(
                    tl_hbm.at[sidx.at[r]],
                    bufls[p].at[pl.ds(i * 128, 128)], semls[p]))
                cps.append(pltpu.async_copy(
                    tr_hbm.at[didx.at[r]],
                    bufrs[p].at[pl.ds(i * 128, 128)], semrs[p]))
            return cps

        infl = {0: fire(0, 0), 1: fire(1, 1)}
        for j in range(CH):
            p = j % 2
            for cp in infl[p]:
                cp.wait()
            off = pl.multiple_of(wid * EW + j * KC, 8)
            pltpu.sync_copy(bufls[p], out_hbm.at[pl.ds(off, KC), pl.ds(0, dt)])
            pltpu.sync_copy(bufrs[p], out_hbm.at[pl.ds(off, KC), pl.ds(dt, dt)])
            if j + 2 < CH:
                infl[p] = fire(j + 2, p)

    return body(src2d, dst2d, tl, tr)


def _scatter_add(dst2d, msg, zrows):
    w = msg.shape[1]
    rt = N_PAD // NS

    @functools.partial(
        pl.kernel,
        out_type=jax.ShapeDtypeStruct((NSC, N_PAD, w), jnp.float32),
        mesh=_sc_mesh(),
        compiler_params=_SC_PARAMS,
        scratch_types=[
            pltpu.VMEM_SHARED((N_PAD, w), jnp.float32),
            pltpu.VMEM((KC, w), jnp.float32),
            pltpu.VMEM((KC, w), jnp.float32),
            pltpu.VMEM((SROWS, 128), jnp.int32),
            pltpu.SemaphoreType.DMA,
            pltpu.SemaphoreType.DMA,
        ],
    )
    def body(dst_hbm, msg_hbm, z_hbm, out_hbm, acc, mbuf0, mbuf1, didx,
             semm0, semm1):
        c = lax.axis_index("c")
        s = lax.axis_index("s")
        wid = s * NSC + c
        base8 = pl.multiple_of((wid * RPT) // 8 * 8, 8)
        o = wid * RPT - base8
        pltpu.sync_copy(dst_hbm.at[pl.ds(base8, SROWS)], didx)
        mbufs, semms = [mbuf0, mbuf1], [semm0, semm1]

        def fire(j, p):
            off = pl.multiple_of(wid * EW + j * KC, 8)
            return pltpu.async_copy(
                msg_hbm.at[pl.ds(off, KC)], mbufs[p], semms[p])

        infl = {0: fire(0, 0), 1: fire(1, 1)}
        row0 = pl.multiple_of(s * rt, 8)
        pltpu.sync_copy(z_hbm, acc.at[pl.ds(row0, rt)])
        plsc.subcore_barrier()

        for j in range(CH):
            p = j % 2
            infl[p].wait()
            for i in range(KR):
                pltpu.sync_copy(mbufs[p].at[pl.ds(i * 128, 128)],
                                acc.at[didx.at[o + j * KR + i]], add=True)
            if j + 2 < CH:
                infl[p] = fire(j + 2, p)
        plsc.subcore_barrier()
        pltpu.sync_copy(acc.at[pl.ds(row0, rt)],
                        out_hbm.at[c, pl.ds(row0, rt)])

    return body(dst2d, msg, zrows)


def _edge_phase2_sc(src2d, dst2d, tl2, tr2, params, zrows):
    w = 16
    rt = N_PAD // NS

    @functools.partial(
        pl.kernel,
        out_type=jax.ShapeDtypeStruct((NSC, N_PAD, w), jnp.float32),
        mesh=_sc_mesh(),
        compiler_params=_SC_PARAMS,
        scratch_types=[
            pltpu.VMEM_SHARED((N_PAD, w), jnp.float32),
            pltpu.VMEM((SROWS, 128), jnp.int32),
            pltpu.VMEM((SROWS, 128), jnp.int32),
            pltpu.VMEM((KC, w), jnp.float32),
            pltpu.VMEM((KC, w), jnp.float32),
            pltpu.VMEM((KC, w), jnp.float32),
            pltpu.VMEM((KC, w), jnp.float32),
            pltpu.VMEM((KC, w), jnp.float32),
            pltpu.VMEM((KC, w), jnp.float32),
            pltpu.VMEM((8, 16), jnp.float32),
            pltpu.SemaphoreType.DMA,
            pltpu.SemaphoreType.DMA,
            pltpu.SemaphoreType.DMA,
            pltpu.SemaphoreType.DMA,
        ],
    )
    def body(src_hbm, dst_hbm, tl_hbm, tr_hbm, par_hbm, z_hbm, out_hbm,
             acc, sidx, didx, bl0, bl1, br0, br1, mb0, mb1, pv,
             seml0, seml1, semr0, semr1):
        c = lax.axis_index("c")
        s = lax.axis_index("s")
        wid = s * NSC + c
        base8 = pl.multiple_of((wid * RPT) // 8 * 8, 8)
        o = wid * RPT - base8
        pltpu.sync_copy(src_hbm.at[pl.ds(base8, SROWS)], sidx)
        pltpu.sync_copy(dst_hbm.at[pl.ds(base8, SROWS)], didx)
        pltpu.sync_copy(par_hbm, pv)
        row0 = pl.multiple_of(s * rt, 8)
        pltpu.sync_copy(z_hbm, acc.at[pl.ds(row0, rt)])
        plsc.subcore_barrier()

        attv = pv[0, :]
        m2v = pv[1, :]
        lane = lax.iota(jnp.int32, 16)
        mask8 = lane < 8
        perms = [jnp.bitwise_xor(lane, r) for r in (1, 2, 4, 8)]
        bls, brs, mbs = [bl0, bl1], [br0, br1], [mb0, mb1]
        semls, semrs = [seml0, seml1], [semr0, semr1]

        def fire(j, p):
            r = o + j * KR
            return (pltpu.async_copy(tl_hbm.at[sidx.at[r]], bls[p], semls[p]),
                    pltpu.async_copy(tr_hbm.at[didx.at[r]], brs[p], semrs[p]))

        def compute(p):
            def step(i, carry):
                hl = bls[p][i, :]
                e = hl + brs[p][i, :]
                e = jnp.where(e > 0, e, 0.2 * e)
                q = e * attv
                for pr in perms:
                    q = q + _take16(q, pr)
                a = jnp.exp(q - m2v)
                mbs[p][i, :] = a * jnp.where(mask8, hl, 1.0)
                return carry
            lax.fori_loop(0, KC, step, 0)

        infl = {0: fire(0, 0), 1: fire(1, 1)}
        for j in range(CH):
            p = j % 2
            for d in infl[p]:
                d.wait()
            compute(p)
            pltpu.sync_copy(mbs[p], acc.at[didx.at[o + j * KR]], add=True)
            if j + 2 < CH:
                infl[p] = fire(j + 2, p)
        plsc.subcore_barrier()
        pltpu.sync_copy(acc.at[pl.ds(row0, rt)],
                        out_hbm.at[c, pl.ds(row0, rt)])

    return body(src2d, dst2d, tl2, tr2, params, zrows)


def kernel(x, edge_index, W1l, b1l, W1r, b1r, att1, bias1,
           W2l, b2l, W2r, b2r, att2, bias2):
    ei = edge_index.astype(jnp.int32)
    loop = jnp.arange(N, dtype=jnp.int32)
    npad = EROWS_PAD * 128 - (ei.shape[1] + N)
    src = jnp.concatenate([ei[0], loop, jnp.zeros((npad,), jnp.int32)])
    dst = jnp.concatenate([ei[1], loop, jnp.full((npad,), N, jnp.int32)])
    src2d = src.reshape(EROWS_PAD, 128)
    dst2d = dst.reshape(EROWS_PAD, 128)

    wcat = jnp.concatenate([W1l, W1r], axis=1)
    bcat = jnp.concatenate([b1l, b1r]).reshape(1, 2 * HC)
    proj, pmn, pmx = _project(x, wcat, bcat)
    xl = proj[:, :HC]
    xr_pad = jnp.concatenate(
        [proj[:, HC:], jnp.zeros((N_PAD - N, HC), jnp.float32)], axis=0)

    sel = (jnp.arange(HC)[:, None] // 8 == jnp.arange(8)[None, :])
    sel = sel.astype(jnp.float32)
    st1 = sel.T

    def lrelu(z):
        return jnp.where(z > 0, z, 0.2 * z)

    af1 = att1.reshape(HC)
    lo1 = lrelu(pmn[0, :HC] + pmn[0, HC:])
    hi1 = lrelu(pmx[0, :HC] + pmx[0, HC:])
    m1 = jnp.maximum(af1 * hi1, af1 * lo1) @ sel
    mx1 = jnp.broadcast_to(m1[None, :], (8, 8))

    xlr = _gather_pairs_combined(src2d, dst2d, xl, xr_pad)
    atts1 = af1[:, None] * sel
    msg1 = _edge_fused1(xlr, atts1, mx1, st1)
    acc1 = _scatter_add(dst2d, msg1, jnp.zeros((N_PAD // NS, 2 * HC), jnp.float32))

    w2cat = jnp.zeros((HC, 32), jnp.float32)
    w2cat = w2cat.at[:, 0:NCLS].set(W2l).at[:, 16:16 + NCLS].set(W2r)
    b2cat = jnp.zeros((32,), jnp.float32)
    b2cat = b2cat.at[0:NCLS].set(b2l).at[16:16 + NCLS].set(b2r)
    tl2, tr2, cmn, cmx = _combine1(acc1[0], acc1[1],
                                   bias1.reshape(1, HC), w2cat,
                                   b2cat.reshape(1, 32))

    att2p = jnp.zeros((16,), jnp.float32).at[0:NCLS].set(att2.reshape(NCLS))
    lo2 = lrelu(cmn[0, 0:16] + cmn[0, 16:32])
    hi2 = lrelu(cmx[0, 0:16] + cmx[0, 16:32])
    m2 = jnp.sum(jnp.maximum(att2p * hi2, att2p * lo2))
    mx2 = jnp.full((8, 8), m2, jnp.float32)

    params2 = (jnp.zeros((8, 16), jnp.float32)
               .at[0].set(att2p)
               .at[1].set(jnp.full((16,), m2, jnp.float32)))
    acc2 = _edge_phase2_sc(src2d, dst2d, tl2, tr2, params2,
                           jnp.zeros((N_PAD // NS, 16), jnp.float32))

    return _final_out(acc2[0], acc2[1], bias2.reshape(1, NCLS))
